# Initial kernel scaffold; baseline (speedup 1.0000x reference)
import jax
import jax.numpy as jnp
from jax.experimental import pallas as pl


def kernel(input, rel_emb, W, a, adj_indices, adj_values):
    raise NotImplementedError("write your pallas kernel here")


# speedup vs baseline: 6.8373x; 6.8373x over previous
import functools

import jax
import jax.numpy as jnp
from jax import lax
from jax.experimental import pallas as pl
from jax.experimental.pallas import tpu as pltpu
from jax.experimental.pallas import tpu_sc as plsc

N = 10000
NP = 10240
FI = 128
FO = 128
FW = 128
E = 320000
E2 = 2 * E
NC, NS, L = 2, 16, 16
NW = NC * NS
G = 128
SUP = 2048
PER_W = 20480
E2P = NW * PER_W
NSUP = PER_W // SUP
ALPHA = 0.2

_MM_BLK = 256
_FIN_BLK = 512


def _mm_body(x_ref, w_ref, a2d_ref, h_ref, pq_ref):
    hb = jnp.dot(x_ref[...], w_ref[...], preferred_element_type=jnp.float32,
                 precision=lax.Precision.HIGHEST)
    pq = jnp.dot(hb, a2d_ref[...], preferred_element_type=jnp.float32,
                 precision=lax.Precision.HIGHEST)
    h_ref[...] = hb
    pq_ref[...] = pq


_mm_call = pl.pallas_call(
    _mm_body,
    grid=(NP // _MM_BLK,),
    in_specs=[
        pl.BlockSpec((_MM_BLK, FI), lambda i: (i, 0)),
        pl.BlockSpec((FI, FO), lambda i: (0, 0)),
        pl.BlockSpec((FO, 2), lambda i: (0, 0)),
    ],
    out_specs=[
        pl.BlockSpec((_MM_BLK, FW), lambda i: (i, 0)),
        pl.BlockSpec((_MM_BLK, 2), lambda i: (i, 0)),
    ],
    out_shape=[
        jax.ShapeDtypeStruct((NP, FW), jnp.float32),
        jax.ShapeDtypeStruct((NP, 2), jnp.float32),
    ],
)


def _rdot_body(re_ref, a2_ref, t_ref):
    rd = jnp.dot(re_ref[...], a2_ref[...], preferred_element_type=jnp.float32,
                 precision=lax.Precision.HIGHEST)
    t_ref[...] = jnp.concatenate([-rd, rd], axis=0)


_rdot_call = pl.pallas_call(
    _rdot_body,
    out_shape=jax.ShapeDtypeStruct((32, 1), jnp.float32),
)


def _fin_body(o_ref, rs_ref, h_ref, pq_ref, out_ref):
    z = pq_ref[:, 0:1] + pq_ref[:, 1:2]
    eye = jnp.exp(-jnp.maximum(z, ALPHA * z))
    num = o_ref[0] + o_ref[1] + eye * h_ref[...]
    den = rs_ref[0] + rs_ref[1] + eye
    out_ref[...] = jnp.maximum(num / den, 0.0)


_fin_call = pl.pallas_call(
    _fin_body,
    grid=(NP // _FIN_BLK,),
    in_specs=[
        pl.BlockSpec((NC, _FIN_BLK, FW), lambda i: (0, i, 0)),
        pl.BlockSpec((NC, _FIN_BLK, 1), lambda i: (0, i, 0)),
        pl.BlockSpec((_FIN_BLK, FW), lambda i: (i, 0)),
        pl.BlockSpec((_FIN_BLK, 2), lambda i: (i, 0)),
    ],
    out_specs=pl.BlockSpec((_FIN_BLK, FO), lambda i: (i, 0)),
    out_shape=jax.ShapeDtypeStruct((NP, FO), jnp.float32),
)


_sc_mesh = plsc.VectorSubcoreMesh(
    core_axis_name="c", subcore_axis_name="s", num_cores=NC, num_subcores=NS)


@functools.partial(
    pl.kernel,
    out_type=[jax.ShapeDtypeStruct((NC, NP, FW), jnp.float32),
              jax.ShapeDtypeStruct((NC, NP), jnp.float32)],
    mesh=_sc_mesh,
    compiler_params=pltpu.CompilerParams(
        needs_layout_passes=False, use_tc_tiling_on_sc=False),
    scratch_types=[
        pltpu.VMEM((NP,), jnp.float32),
        pltpu.VMEM((NP,), jnp.float32),
        pltpu.VMEM((32,), jnp.float32),
        pltpu.VMEM((SUP // G, G), jnp.int32),
        pltpu.VMEM((SUP // G, G), jnp.int32),
        pltpu.VMEM((SUP // G, G), jnp.int32),
        pltpu.VMEM((SUP // G, G), jnp.float32),
        pltpu.VMEM((G, FW), jnp.float32),
        pltpu.VMEM_SHARED((NP, FW), jnp.float32),
        pltpu.VMEM_SHARED((NP,), jnp.float32),
        pltpu.SemaphoreType.DMA,
    ],
)
def _sc_edges(h_hbm, p_hbm, q_hbm, t_hbm, s_hbm, d_hbm, v_hbm, z_hbm, z1_hbm,
              out_hbm, rs_hbm, p_v, q_v, t_v, s_v, d_v, v_v, e_v, hr_v,
              acc_sh, rs_sh, sem):
    c = lax.axis_index("c")
    s = lax.axis_index("s")
    wid = c * NS + s

    pltpu.sync_copy(p_hbm, p_v)
    pltpu.sync_copy(q_hbm, q_v)
    pltpu.sync_copy(t_hbm, t_v)

    rows_per_tile = NP // NS
    row0 = s * rows_per_tile
    for k in range(rows_per_tile // G):
        pltpu.sync_copy(z_hbm, acc_sh.at[pl.ds(row0 + k * G, G)])
    pltpu.sync_copy(z1_hbm.at[pl.ds(row0, rows_per_tile)],
                    rs_sh.at[pl.ds(row0, rows_per_tile)])
    plsc.subcore_barrier()

    rowbase = wid * (PER_W // G)

    def sup_body(i, carry):
        rowb = rowbase + i * (SUP // G)
        pltpu.sync_copy(s_hbm.at[pl.ds(rowb, SUP // G)], s_v)
        pltpu.sync_copy(d_hbm.at[pl.ds(rowb, SUP // G)], d_v)
        pltpu.sync_copy(v_hbm.at[pl.ds(rowb, SUP // G)], v_v)

        def group_body(j, carry2):
            pltpu.async_copy(h_hbm.at[d_v.at[j]], hr_v, sem).wait()
            for k in range(G // L):
                sl = pl.ds(k * L, L)
                sv = s_v[j, sl]
                dv = d_v[j, sl]
                vv = v_v[j, sl]
                pv = plsc.load_gather(p_v, [sv])
                qv = plsc.load_gather(q_v, [dv])
                cv = plsc.load_gather(t_v, [vv])
                zz = pv + qv + cv
                e_v[j, sl] = jnp.exp(-jnp.maximum(zz, ALPHA * zz))

            def scale_body(rb, carry3):
                e16 = e_v[j, pl.ds(rb * L, L)]
                for rr in range(L):
                    r = rb * L + rr
                    ej = jnp.full((L,), e16[rr], jnp.float32)
                    for f in range(FW // L):
                        fs = pl.ds(f * L, L)
                        hr_v[r, fs] = hr_v[r, fs] * ej
                return carry3
            lax.fori_loop(0, G // L, scale_body, 0)

            pltpu.sync_copy(hr_v, acc_sh.at[s_v.at[j]], add=True)
            pltpu.sync_copy(e_v.at[j], rs_sh.at[s_v.at[j]], add=True)
            return carry2
        lax.fori_loop(0, SUP // G, group_body, 0)
        return carry
    lax.fori_loop(0, NSUP, sup_body, 0)

    plsc.subcore_barrier()
    pltpu.sync_copy(acc_sh.at[pl.ds(row0, rows_per_tile)],
                    out_hbm.at[c, pl.ds(row0, rows_per_tile)])
    pltpu.sync_copy(rs_sh.at[pl.ds(row0, rows_per_tile)],
                    rs_hbm.at[c, pl.ds(row0, rows_per_tile)])


def kernel(input, rel_emb, W, a, adj_indices, adj_values):
    x = input.astype(jnp.float32)
    xp = jnp.zeros((NP, FI), jnp.float32).at[:N].set(x)
    a2d = a.reshape(2, FO).T
    a2col = a2d[:, 1:2]

    h_sc, pq = _mm_call(xp, W, a2d)
    t = _rdot_call(rel_emb, a2col).reshape(32)

    src = adj_indices[0]
    dst = adj_indices[1]
    pad = E2P - E2
    S = jnp.concatenate([src, dst, jnp.full((pad,), NP - 1, jnp.int32)])
    D = jnp.concatenate([dst, src, jnp.full((pad,), NP - 1, jnp.int32)])
    V = jnp.concatenate([adj_values, adj_values + 16,
                         jnp.zeros((pad,), jnp.int32)])
    S2 = S.reshape(E2P // G, G)
    D2 = D.reshape(E2P // G, G)
    V2 = V.reshape(E2P // G, G)

    p = pq[:, 0]
    q = pq[:, 1]
    zeros_blk = jnp.zeros((G, FW), jnp.float32)
    zeros_row = jnp.zeros((NP,), jnp.float32)

    out01, rs01 = _sc_edges(h_sc, p, q, t, S2, D2, V2, zeros_blk, zeros_row)
    res = _fin_call(out01, rs01.reshape(NC, NP, 1), h_sc, pq)
    return res[:N]


# speedup vs baseline: 8.9960x; 1.3157x over previous
import functools

import jax
import jax.numpy as jnp
import numpy as np
from jax import lax
from jax.experimental import pallas as pl
from jax.experimental.pallas import tpu as pltpu
from jax.experimental.pallas import tpu_sc as plsc

N = 10000
NP = 10240
FI = 128
FO = 128
FW = 128
E = 320000
E2 = 2 * E
NC, NS, L = 2, 16, 16
NW = NC * NS
G = 64
SUP = 1024
PER_W = 20480
E2P = NW * PER_W
NSUP = PER_W // SUP
ALPHA = 0.2

_MM_BLK = 256
_FIN_BLK = 512


def _mm_body(x_ref, w_ref, wp_ref, a2d_ref, h_ref, hb16_ref, pq_ref):
    hb = jnp.dot(x_ref[...], w_ref[...], preferred_element_type=jnp.float32,
                 precision=lax.Precision.HIGHEST)
    hbp = jnp.dot(x_ref[...], wp_ref[...], preferred_element_type=jnp.float32,
                  precision=lax.Precision.HIGHEST)
    pq = jnp.dot(hb, a2d_ref[...], preferred_element_type=jnp.float32,
                 precision=lax.Precision.HIGHEST)
    h_ref[...] = hb
    hb16_ref[...] = hbp.astype(jnp.bfloat16)
    pq_ref[...] = pq


_mm_call = pl.pallas_call(
    _mm_body,
    grid=(NP // _MM_BLK,),
    in_specs=[
        pl.BlockSpec((_MM_BLK, FI), lambda i: (i, 0)),
        pl.BlockSpec((FI, FO), lambda i: (0, 0)),
        pl.BlockSpec((FI, FO), lambda i: (0, 0)),
        pl.BlockSpec((FO, 2), lambda i: (0, 0)),
    ],
    out_specs=[
        pl.BlockSpec((_MM_BLK, FW), lambda i: (i, 0)),
        pl.BlockSpec((_MM_BLK, FW), lambda i: (i, 0)),
        pl.BlockSpec((_MM_BLK, 2), lambda i: (i, 0)),
    ],
    out_shape=[
        jax.ShapeDtypeStruct((NP, FW), jnp.float32),
        jax.ShapeDtypeStruct((NP, FW), jnp.bfloat16),
        jax.ShapeDtypeStruct((NP, 2), jnp.float32),
    ],
)

_IPP = np.empty(FO, np.int32)
for _c in range(FO // 32):
    for _r in range(16):
        _IPP[32 * _c + 2 * _r] = 32 * _c + _r
        _IPP[32 * _c + 2 * _r + 1] = 32 * _c + 16 + _r


def _rdot_body(re_ref, a2_ref, t_ref):
    rd = jnp.dot(re_ref[...], a2_ref[...], preferred_element_type=jnp.float32,
                 precision=lax.Precision.HIGHEST)
    t_ref[...] = jnp.concatenate([-rd, rd], axis=0)


_rdot_call = pl.pallas_call(
    _rdot_body,
    out_shape=jax.ShapeDtypeStruct((32, 1), jnp.float32),
)


def _fin_body(o_ref, rs_ref, h_ref, pq_ref, out_ref):
    z = pq_ref[:, 0:1] + pq_ref[:, 1:2]
    eye = jnp.exp(-jnp.maximum(z, ALPHA * z))
    num = o_ref[0] + o_ref[1] + eye * h_ref[...]
    den = rs_ref[0] + rs_ref[1] + eye
    out_ref[...] = jnp.maximum(num / den, 0.0)


_fin_call = pl.pallas_call(
    _fin_body,
    grid=(NP // _FIN_BLK,),
    in_specs=[
        pl.BlockSpec((NC, _FIN_BLK, FW), lambda i: (0, i, 0)),
        pl.BlockSpec((NC, _FIN_BLK, 1), lambda i: (0, i, 0)),
        pl.BlockSpec((_FIN_BLK, FW), lambda i: (i, 0)),
        pl.BlockSpec((_FIN_BLK, 2), lambda i: (i, 0)),
    ],
    out_specs=pl.BlockSpec((_FIN_BLK, FO), lambda i: (i, 0)),
    out_shape=jax.ShapeDtypeStruct((NP, FO), jnp.float32),
)


_sc_mesh = plsc.VectorSubcoreMesh(
    core_axis_name="c", subcore_axis_name="s", num_cores=NC, num_subcores=NS)


@functools.partial(
    pl.kernel,
    out_type=[jax.ShapeDtypeStruct((NC, NP, FW), jnp.float32),
              jax.ShapeDtypeStruct((NC, NP), jnp.float32)],
    mesh=_sc_mesh,
    compiler_params=pltpu.CompilerParams(
        needs_layout_passes=False, use_tc_tiling_on_sc=False),
    scratch_types=[
        pltpu.VMEM((NP,), jnp.float32),
        pltpu.VMEM((NP,), jnp.float32),
        pltpu.VMEM((32,), jnp.float32),
        pltpu.VMEM((SUP // G, G), jnp.int32),
        pltpu.VMEM((SUP // G, G), jnp.int32),
        pltpu.VMEM((SUP // G, G), jnp.int32),
        pltpu.VMEM((SUP // G, G + L), jnp.float32),
        pltpu.VMEM((G, FW), jnp.bfloat16),
        pltpu.VMEM((G, FW), jnp.bfloat16),
        pltpu.VMEM((G, FW), jnp.float32),
        pltpu.VMEM_SHARED((NP, FW), jnp.float32),
        pltpu.VMEM_SHARED((NP,), jnp.float32),
        pltpu.SemaphoreType.DMA,
        pltpu.SemaphoreType.DMA,
    ],
)
def _sc_edges(h_hbm, p_hbm, q_hbm, t_hbm, s_hbm, d_hbm, v_hbm, z_hbm, z1_hbm,
              out_hbm, rs_hbm, p_v, q_v, t_v, s_v, d_v, v_v, e_v,
              hrb0, hrb1, hr_v, acc_sh, rs_sh, sg0, sg1):
    c = lax.axis_index("c")
    s = lax.axis_index("s")
    wid = c * NS + s
    hrb = (hrb0, hrb1)
    sg = (sg0, sg1)

    pltpu.sync_copy(p_hbm, p_v)
    pltpu.sync_copy(q_hbm, q_v)
    pltpu.sync_copy(t_hbm, t_v)

    rows_per_tile = NP // NS
    row0 = s * rows_per_tile
    for k in range(rows_per_tile // G):
        pltpu.sync_copy(z_hbm, acc_sh.at[pl.ds(row0 + k * G, G)])
    pltpu.sync_copy(z1_hbm.at[pl.ds(row0, rows_per_tile)],
                    rs_sh.at[pl.ds(row0, rows_per_tile)])

    rowbase = wid * (PER_W // G)
    pltpu.sync_copy(s_hbm.at[pl.ds(rowbase, SUP // G)], s_v)
    pltpu.sync_copy(d_hbm.at[pl.ds(rowbase, SUP // G)], d_v)
    pltpu.sync_copy(v_hbm.at[pl.ds(rowbase, SUP // G)], v_v)
    plsc.subcore_barrier()

    def sup_body(i, carry):
        pltpu.async_copy(h_hbm.at[d_v.at[0]], hrb0, sg0)
        for j in range(SUP // G):
            b = j & 1
            if j + 1 < SUP // G:
                pltpu.async_copy(h_hbm.at[d_v.at[j + 1]], hrb[1 - b],
                                 sg[1 - b])

            def e_body(k, carry2, _j=j):
                sl = pl.ds(k * L, L)
                sv = s_v[_j, sl]
                dv = d_v[_j, sl]
                vv = v_v[_j, sl]
                pv = plsc.load_gather(p_v, [sv])
                qv = plsc.load_gather(q_v, [dv])
                cv = plsc.load_gather(t_v, [vv])
                zz = pv + qv + cv
                e_v[_j, sl] = jnp.exp(-jnp.maximum(zz, ALPHA * zz))
                return carry2
            lax.fori_loop(0, G // L, e_body, 0)

            pltpu.make_async_copy(h_hbm.at[d_v.at[j]], hrb[b], sg[b]).wait()

            def scale_body(rb, carry3, _j=j, _b=b):
                r0 = rb * 8
                e16 = e_v[_j, pl.ds(r0, L)]
                for rr in range(8):
                    r = r0 + rr
                    ej = jnp.full((L,), e16[rr], jnp.float32)
                    for cc in range(FW // 32):
                        ab = hrb[_b][r, pl.ds(cc * 32, 32)]
                        lo, hi = plsc.unpack(
                            ab, format=plsc.PackFormat.INTERLEAVED)
                        hr_v[r, pl.ds(cc * 32, L)] = lo * ej
                        hr_v[r, pl.ds(cc * 32 + L, L)] = hi * ej
                return carry3
            lax.fori_loop(0, G // 8, scale_body, 0)

            pltpu.sync_copy(hr_v, acc_sh.at[s_v.at[j]], add=True)
            pltpu.sync_copy(e_v.at[j, pl.ds(0, G)], rs_sh.at[s_v.at[j]],
                            add=True)

        @pl.when(i < NSUP - 1)
        def _():
            rowb = rowbase + (i + 1) * (SUP // G)
            pltpu.sync_copy(s_hbm.at[pl.ds(rowb, SUP // G)], s_v)
            pltpu.sync_copy(d_hbm.at[pl.ds(rowb, SUP // G)], d_v)
            pltpu.sync_copy(v_hbm.at[pl.ds(rowb, SUP // G)], v_v)
        return carry
    lax.fori_loop(0, NSUP, sup_body, 0)

    plsc.subcore_barrier()
    pltpu.sync_copy(acc_sh.at[pl.ds(row0, rows_per_tile)],
                    out_hbm.at[c, pl.ds(row0, rows_per_tile)])
    pltpu.sync_copy(rs_sh.at[pl.ds(row0, rows_per_tile)],
                    rs_hbm.at[c, pl.ds(row0, rows_per_tile)])


def kernel(input, rel_emb, W, a, adj_indices, adj_values):
    x = input.astype(jnp.float32)
    xp = jnp.zeros((NP, FI), jnp.float32).at[:N].set(x)
    a2d = a.reshape(2, FO).T
    a2col = a2d[:, 1:2]
    wp = W[:, _IPP]

    h_sc, hb16, pq = _mm_call(xp, W, wp, a2d)
    t = _rdot_call(rel_emb, a2col).reshape(32)

    src = adj_indices[0]
    dst = adj_indices[1]
    pad = E2P - E2
    S = jnp.concatenate([src, dst, jnp.full((pad,), NP - 1, jnp.int32)])
    D = jnp.concatenate([dst, src, jnp.full((pad,), NP - 1, jnp.int32)])
    V = jnp.concatenate([adj_values, adj_values + 16,
                         jnp.zeros((pad,), jnp.int32)])
    S2 = S.reshape(E2P // G, G)
    D2 = D.reshape(E2P // G, G)
    V2 = V.reshape(E2P // G, G)

    p = pq[:, 0]
    q = pq[:, 1]
    zeros_blk = jnp.zeros((G, FW), jnp.float32)
    zeros_row = jnp.zeros((NP,), jnp.float32)

    out01, rs01 = _sc_edges(hb16, p, q, t, S2, D2, V2, zeros_blk, zeros_row)
    res = _fin_call(out01, rs01.reshape(NC, NP, 1), h_sc, pq)
    return res[:N]
